# Initial kernel scaffold; baseline (speedup 1.0000x reference)
import jax
import jax.numpy as jnp
from jax.experimental import pallas as pl


def kernel(x, edge_index, batch, fh_W1, fh_b1, fh_g1, fh_be1, fh_W2, fh_b2, fh_g2, fh_be2, c1_W1, c1_b1, c1_g1, c1_be1, c1_W2, c1_b2, c1_g2, c1_be2, c2_W1, c2_b1, c2_g1, c2_be1, c2_W2, c2_b2, c2_g2, c2_be2, lin_W0, lin_b0, lin_W1, lin_b1, lin_W2, lin_b2):
    raise NotImplementedError("write your pallas kernel here")


# speedup vs baseline: 4.4102x; 4.4102x over previous
import functools

import jax
import jax.numpy as jnp
from jax import lax
from jax.experimental import pallas as pl
from jax.experimental.pallas import tpu as pltpu
from jax.experimental.pallas import tpu_sc as plsc

N = 10000
E = 320000
DF = 128
H = 64
T = 10
G = 64
HP = 128

NC = 2
NS = 16
NW = NC * NS
EPW = E // NW
CHUNK = 80
NCHUNK = EPW // CHUNK
RPT = 624
TAIL = N - NS * RPT


def _bn_relu(h, g, b):
    m = jnp.mean(h, axis=0, keepdims=True)
    v = jnp.mean((h - m) * (h - m), axis=0, keepdims=True)
    return jax.nn.relu((h - m) / jnp.sqrt(v + 1e-5) * g + b)


def _mlp(h, w1, b1, g1, be1, w2, b2, g2, be2):
    h = jnp.dot(h, w1, preferred_element_type=jnp.float32) + b1
    h = _bn_relu(h, g1, be1)
    h = jnp.dot(h, w2, preferred_element_type=jnp.float32) + b2
    h = _bn_relu(h, g2, be2)
    return h


def _pool(h, batch_row):
    onehot = (lax.broadcasted_iota(jnp.int32, (G, N), 0) == batch_row)
    onehot = onehot.astype(jnp.float32)
    pool = jnp.dot(onehot, h, preferred_element_type=jnp.float32)
    cnt = jnp.sum(onehot, axis=1, keepdims=True)
    return pool, cnt


def _first_body(x_ref, b_ref, w1_ref, b1_ref, g1_ref, be1_ref,
                w2_ref, b2_ref, g2_ref, be2_ref, lw_ref, lb_ref,
                h_ref, out_ref):
    h = _mlp(x_ref[...], w1_ref[...], b1_ref[...], g1_ref[...], be1_ref[...],
             w2_ref[...], b2_ref[...], g2_ref[...], be2_ref[...])
    h_ref[...] = jnp.concatenate([h, jnp.zeros_like(h)], axis=1)
    pool, cnt = _pool(h, b_ref[...])
    out_ref[...] = (jnp.dot(pool, lw_ref[...], preferred_element_type=jnp.float32)
                    + cnt * lb_ref[...])


def _layer_body(x_ref, agg_ref, b_ref, w1_ref, b1_ref, g1_ref, be1_ref,
                w2_ref, b2_ref, g2_ref, be2_ref, lw_ref, lb_ref, oin_ref,
                h_ref, out_ref):
    a = agg_ref[...]
    xin = (x_ref[...] + a[:N] + a[N:])[:, :H]
    h = _mlp(xin, w1_ref[...], b1_ref[...], g1_ref[...], be1_ref[...],
             w2_ref[...], b2_ref[...], g2_ref[...], be2_ref[...])
    h_ref[...] = jnp.concatenate([h, jnp.zeros_like(h)], axis=1)
    pool, _ = _pool(h, b_ref[...])
    out_ref[...] = (oin_ref[...]
                    + jnp.dot(pool, lw_ref[...], preferred_element_type=jnp.float32)
                    + lb_ref[...])


_mlp_first = pl.pallas_call(
    _first_body,
    out_shape=[jax.ShapeDtypeStruct((N, HP), jnp.float32),
               jax.ShapeDtypeStruct((G, T), jnp.float32)],
)

_mlp_layer = pl.pallas_call(
    _layer_body,
    out_shape=[jax.ShapeDtypeStruct((N, HP), jnp.float32),
               jax.ShapeDtypeStruct((G, T), jnp.float32)],
)


def _edge_agg_body(x_hbm, src_hbm, dst_hbm, zero_hbm, out_hbm,
                   sidx, didx, rows, acc, sem):
    cid = lax.axis_index("c")
    sid = lax.axis_index("s")
    wid = sid * NC + cid
    pltpu.sync_copy(zero_hbm.at[pl.ds(sid * RPT, RPT)],
                    acc.at[pl.ds(sid * RPT, RPT)])

    @pl.when(sid == NS - 1)
    def _():
        pltpu.sync_copy(zero_hbm.at[pl.ds(NS * RPT, TAIL)],
                        acc.at[pl.ds(NS * RPT, TAIL)])

    plsc.subcore_barrier()
    ebase = wid * EPW

    def body(j, carry):
        off = ebase + j * CHUNK
        pltpu.sync_copy(src_hbm.at[pl.ds(off, CHUNK)], sidx)
        pltpu.sync_copy(dst_hbm.at[pl.ds(off, CHUNK)], didx)
        pltpu.async_copy(x_hbm.at[sidx], rows, sem).wait()
        pltpu.sync_copy(rows, acc.at[didx], add=True)
        return carry

    lax.fori_loop(0, NCHUNK, body, 0)
    plsc.subcore_barrier()
    obase = cid * N + sid * RPT
    pltpu.sync_copy(acc.at[pl.ds(sid * RPT, RPT)],
                    out_hbm.at[pl.ds(obase, RPT)])

    @pl.when(sid == NS - 1)
    def _():
        pltpu.sync_copy(acc.at[pl.ds(NS * RPT, TAIL)],
                        out_hbm.at[pl.ds(cid * N + NS * RPT, TAIL)])


@functools.lru_cache(maxsize=None)
def _edge_agg_kernel():
    return pl.kernel(
        _edge_agg_body,
        out_type=jax.ShapeDtypeStruct((NC * N, HP), jnp.float32),
        mesh=plsc.VectorSubcoreMesh(core_axis_name="c", subcore_axis_name="s",
                                    num_cores=NC, num_subcores=NS),
        scratch_types=[
            pltpu.VMEM((CHUNK,), jnp.int32),
            pltpu.VMEM((CHUNK,), jnp.int32),
            pltpu.VMEM((CHUNK, HP), jnp.float32),
            pltpu.VMEM_SHARED((N, HP), jnp.float32),
            pltpu.SemaphoreType.DMA,
        ],
    )


def _edge_agg(x0, src, dst, zeros):
    return _edge_agg_kernel()(x0, src, dst, zeros)


def kernel(x, edge_index, batch, fh_W1, fh_b1, fh_g1, fh_be1, fh_W2, fh_b2,
           fh_g2, fh_be2, c1_W1, c1_b1, c1_g1, c1_be1, c1_W2, c1_b2, c1_g2,
           c1_be2, c2_W1, c2_b1, c2_g1, c2_be1, c2_W2, c2_b2, c2_g2, c2_be2,
           lin_W0, lin_b0, lin_W1, lin_b1, lin_W2, lin_b2):
    src = edge_index[0]
    dst = edge_index[1]
    brow = batch.reshape(1, N)
    zeros = jnp.zeros((N, HP), jnp.float32)
    r = lambda a: a.reshape(1, -1)

    x0, out = _mlp_first(x, brow, fh_W1, r(fh_b1), r(fh_g1), r(fh_be1),
                         fh_W2, r(fh_b2), r(fh_g2), r(fh_be2),
                         lin_W0, r(lin_b0))
    agg = _edge_agg(x0, src, dst, zeros)
    x1, out = _mlp_layer(x0, agg, brow, c1_W1, r(c1_b1), r(c1_g1), r(c1_be1),
                         c1_W2, r(c1_b2), r(c1_g2), r(c1_be2),
                         lin_W1, r(lin_b1), out)
    agg = _edge_agg(x1, src, dst, zeros)
    _, out = _mlp_layer(x1, agg, brow, c2_W1, r(c2_b1), r(c2_g1), r(c2_be1),
                        c2_W2, r(c2_b2), r(c2_g2), r(c2_be2),
                        lin_W2, r(lin_b2), out)
    return out


# speedup vs baseline: 7.9110x; 1.7938x over previous
import functools

import jax
import jax.numpy as jnp
from jax import lax
from jax.experimental import pallas as pl
from jax.experimental.pallas import tpu as pltpu
from jax.experimental.pallas import tpu_sc as plsc

N = 10000
E = 320000
DF = 128
H = 64
T = 10
G = 64
HP = 128

NC = 2
NS = 16
NW = NC * NS
EPW = E // NW
CHUNK = 80
NCHUNK = EPW // CHUNK
RPT = 624
TAIL = N - NS * RPT


def _bn_relu(h, g, b):
    m = jnp.mean(h, axis=0, keepdims=True)
    v = jnp.mean((h - m) * (h - m), axis=0, keepdims=True)
    return jax.nn.relu((h - m) / jnp.sqrt(v + 1e-5) * g + b)


def _mlp(h, w1, b1, g1, be1, w2, b2, g2, be2):
    h = jnp.dot(h, w1, preferred_element_type=jnp.float32) + b1
    h = _bn_relu(h, g1, be1)
    h = jnp.dot(h, w2, preferred_element_type=jnp.float32) + b2
    h = _bn_relu(h, g2, be2)
    return h


def _pool(h, batch_row):
    onehot = (lax.broadcasted_iota(jnp.int32, (G, N), 0) == batch_row)
    onehot = onehot.astype(jnp.float32)
    pool = jnp.dot(onehot, h, preferred_element_type=jnp.float32)
    cnt = jnp.sum(onehot, axis=1, keepdims=True)
    return pool, cnt


def _first_body(x_ref, b_ref, w1_ref, b1_ref, g1_ref, be1_ref,
                w2_ref, b2_ref, g2_ref, be2_ref, lw_ref, lb_ref,
                h_ref, out_ref):
    h = _mlp(x_ref[...], w1_ref[...], b1_ref[...], g1_ref[...], be1_ref[...],
             w2_ref[...], b2_ref[...], g2_ref[...], be2_ref[...])
    h_ref[...] = jnp.concatenate([h, jnp.zeros_like(h)], axis=1)
    pool, cnt = _pool(h, b_ref[...])
    out_ref[...] = (jnp.dot(pool, lw_ref[...], preferred_element_type=jnp.float32)
                    + cnt * lb_ref[...])


def _layer_body(x_ref, agg_ref, b_ref, w1_ref, b1_ref, g1_ref, be1_ref,
                w2_ref, b2_ref, g2_ref, be2_ref, lw_ref, lb_ref, oin_ref,
                h_ref, out_ref):
    a = agg_ref[...]
    xin = (x_ref[...] + a[:N] + a[N:])[:, :H]
    h = _mlp(xin, w1_ref[...], b1_ref[...], g1_ref[...], be1_ref[...],
             w2_ref[...], b2_ref[...], g2_ref[...], be2_ref[...])
    h_ref[...] = jnp.concatenate([h, jnp.zeros_like(h)], axis=1)
    pool, _ = _pool(h, b_ref[...])
    out_ref[...] = (oin_ref[...]
                    + jnp.dot(pool, lw_ref[...], preferred_element_type=jnp.float32)
                    + lb_ref[...])


_mlp_first = pl.pallas_call(
    _first_body,
    out_shape=[jax.ShapeDtypeStruct((N, HP), jnp.float32),
               jax.ShapeDtypeStruct((G, T), jnp.float32)],
)

_mlp_layer = pl.pallas_call(
    _layer_body,
    out_shape=[jax.ShapeDtypeStruct((N, HP), jnp.float32),
               jax.ShapeDtypeStruct((G, T), jnp.float32)],
)


UNROLL = 4
NITER = (NCHUNK - 1) // UNROLL


def _edge_agg_body(x_hbm, src_hbm, dst_hbm, zero_hbm, out_hbm,
                   sidxs, didxs, rows, acc, gsems, ssems, dsems, qsems):
    cid = lax.axis_index("c")
    sid = lax.axis_index("s")
    wid = sid * NC + cid
    ebase = wid * EPW

    def do_chunks(offs):
        loads = []
        for k, off in enumerate(offs):
            sld = pltpu.async_copy(src_hbm.at[pl.ds(off, CHUNK)],
                                   sidxs[k], qsems[k])
            dld = pltpu.async_copy(dst_hbm.at[pl.ds(off, CHUNK)],
                                   didxs[k], dsems[k])
            loads.append((sld, dld))
        gathers = []
        for k in range(len(offs)):
            loads[k][0].wait()
            g = pltpu.async_copy(x_hbm.at[sidxs[k]], rows[k], gsems[k])
            gathers.append(g)
        scatters = []
        for k in range(len(offs)):
            loads[k][1].wait()
            gathers[k].wait()
            s = pltpu.make_async_copy(rows[k], acc.at[didxs[k]], ssems[k])
            s.start(add=True)
            scatters.append(s)
        for s in scatters:
            s.wait()

    pltpu.sync_copy(zero_hbm.at[pl.ds(sid * RPT, RPT)],
                    acc.at[pl.ds(sid * RPT, RPT)])

    @pl.when(sid == NS - 1)
    def _():
        pltpu.sync_copy(zero_hbm.at[pl.ds(NS * RPT, TAIL)],
                        acc.at[pl.ds(NS * RPT, TAIL)])

    plsc.subcore_barrier()

    def body(i, carry):
        base = ebase + i * (UNROLL * CHUNK)
        do_chunks([base + k * CHUNK for k in range(UNROLL)])
        return carry

    lax.fori_loop(0, NITER, body, 0)
    do_chunks([ebase + NITER * UNROLL * CHUNK])
    plsc.subcore_barrier()
    obase = cid * N + sid * RPT
    pltpu.sync_copy(acc.at[pl.ds(sid * RPT, RPT)],
                    out_hbm.at[pl.ds(obase, RPT)])

    @pl.when(sid == NS - 1)
    def _():
        pltpu.sync_copy(acc.at[pl.ds(NS * RPT, TAIL)],
                        out_hbm.at[pl.ds(cid * N + NS * RPT, TAIL)])


@functools.lru_cache(maxsize=None)
def _edge_agg_kernel():
    return pl.kernel(
        _edge_agg_body,
        out_type=jax.ShapeDtypeStruct((NC * N, HP), jnp.float32),
        mesh=plsc.VectorSubcoreMesh(core_axis_name="c", subcore_axis_name="s",
                                    num_cores=NC, num_subcores=NS),
        scratch_types=[
            [pltpu.VMEM((CHUNK,), jnp.int32) for _ in range(UNROLL)],
            [pltpu.VMEM((CHUNK,), jnp.int32) for _ in range(UNROLL)],
            [pltpu.VMEM((CHUNK, HP), jnp.float32) for _ in range(UNROLL)],
            pltpu.VMEM_SHARED((N, HP), jnp.float32),
            [pltpu.SemaphoreType.DMA for _ in range(UNROLL)],
            [pltpu.SemaphoreType.DMA for _ in range(UNROLL)],
            [pltpu.SemaphoreType.DMA for _ in range(UNROLL)],
            [pltpu.SemaphoreType.DMA for _ in range(UNROLL)],
        ],
    )


def _edge_agg(x0, src, dst, zeros):
    return _edge_agg_kernel()(x0, src, dst, zeros)


def kernel(x, edge_index, batch, fh_W1, fh_b1, fh_g1, fh_be1, fh_W2, fh_b2,
           fh_g2, fh_be2, c1_W1, c1_b1, c1_g1, c1_be1, c1_W2, c1_b2, c1_g2,
           c1_be2, c2_W1, c2_b1, c2_g1, c2_be1, c2_W2, c2_b2, c2_g2, c2_be2,
           lin_W0, lin_b0, lin_W1, lin_b1, lin_W2, lin_b2):
    src = edge_index[0]
    dst = edge_index[1]
    brow = batch.reshape(1, N)
    zeros = jnp.zeros((N, HP), jnp.float32)
    r = lambda a: a.reshape(1, -1)

    x0, out = _mlp_first(x, brow, fh_W1, r(fh_b1), r(fh_g1), r(fh_be1),
                         fh_W2, r(fh_b2), r(fh_g2), r(fh_be2),
                         lin_W0, r(lin_b0))
    agg = _edge_agg(x0, src, dst, zeros)
    x1, out = _mlp_layer(x0, agg, brow, c1_W1, r(c1_b1), r(c1_g1), r(c1_be1),
                         c1_W2, r(c1_b2), r(c1_g2), r(c1_be2),
                         lin_W1, r(lin_b1), out)
    agg = _edge_agg(x1, src, dst, zeros)
    _, out = _mlp_layer(x1, agg, brow, c2_W1, r(c2_b1), r(c2_g1), r(c2_be1),
                        c2_W2, r(c2_b2), r(c2_g2), r(c2_be2),
                        lin_W2, r(lin_b2), out)
    return out


# speedup vs baseline: 7.9731x; 1.0079x over previous
import functools

import jax
import jax.numpy as jnp
from jax import lax
from jax.experimental import pallas as pl
from jax.experimental.pallas import tpu as pltpu
from jax.experimental.pallas import tpu_sc as plsc

N = 10000
E = 320000
DF = 128
H = 64
T = 10
G = 64
HP = 128

NC = 2
NS = 16
NW = NC * NS
EPW = E // NW
CHUNK = 128
NFULL = EPW // CHUNK
TAILE = EPW - NFULL * CHUNK
RPT = 624
TAIL = N - NS * RPT


def _bn_relu(h, g, b):
    m = jnp.mean(h, axis=0, keepdims=True)
    v = jnp.mean((h - m) * (h - m), axis=0, keepdims=True)
    return jax.nn.relu((h - m) / jnp.sqrt(v + 1e-5) * g + b)


def _mlp(h, w1, b1, g1, be1, w2, b2, g2, be2):
    h = jnp.dot(h, w1, preferred_element_type=jnp.float32) + b1
    h = _bn_relu(h, g1, be1)
    h = jnp.dot(h, w2, preferred_element_type=jnp.float32) + b2
    h = _bn_relu(h, g2, be2)
    return h


def _pool(h, batch_row):
    onehot = (lax.broadcasted_iota(jnp.int32, (G, N), 0) == batch_row)
    onehot = onehot.astype(jnp.float32)
    pool = jnp.dot(onehot, h, preferred_element_type=jnp.float32)
    cnt = jnp.sum(onehot, axis=1, keepdims=True)
    return pool, cnt


def _first_body(x_ref, b_ref, w1_ref, b1_ref, g1_ref, be1_ref,
                w2_ref, b2_ref, g2_ref, be2_ref, lw_ref, lb_ref,
                h_ref, out_ref):
    h = _mlp(x_ref[...], w1_ref[...], b1_ref[...], g1_ref[...], be1_ref[...],
             w2_ref[...], b2_ref[...], g2_ref[...], be2_ref[...])
    h_ref[...] = jnp.concatenate([h, jnp.zeros_like(h)], axis=1)
    pool, cnt = _pool(h, b_ref[...])
    out_ref[...] = (jnp.dot(pool, lw_ref[...], preferred_element_type=jnp.float32)
                    + cnt * lb_ref[...])


def _layer_body(x_ref, agg_ref, b_ref, w1_ref, b1_ref, g1_ref, be1_ref,
                w2_ref, b2_ref, g2_ref, be2_ref, lw_ref, lb_ref, oin_ref,
                h_ref, out_ref):
    a = agg_ref[...]
    xin = (x_ref[...] + a[:N] + a[N:])[:, :H]
    h = _mlp(xin, w1_ref[...], b1_ref[...], g1_ref[...], be1_ref[...],
             w2_ref[...], b2_ref[...], g2_ref[...], be2_ref[...])
    h_ref[...] = jnp.concatenate([h, jnp.zeros_like(h)], axis=1)
    pool, _ = _pool(h, b_ref[...])
    out_ref[...] = (oin_ref[...]
                    + jnp.dot(pool, lw_ref[...], preferred_element_type=jnp.float32)
                    + lb_ref[...])


_mlp_first = pl.pallas_call(
    _first_body,
    out_shape=[jax.ShapeDtypeStruct((N, HP), jnp.float32),
               jax.ShapeDtypeStruct((G, T), jnp.float32)],
)

_mlp_layer = pl.pallas_call(
    _layer_body,
    out_shape=[jax.ShapeDtypeStruct((N, HP), jnp.float32),
               jax.ShapeDtypeStruct((G, T), jnp.float32)],
)


UNROLL = 3
NITER = NFULL // UNROLL


def _edge_agg_body(x_hbm, src_hbm, dst_hbm, zero_hbm, out_hbm,
                   sidxs, didxs, tsidx, tdidx, rows, acc,
                   gsems, ssems, dsems, qsems, tsem):
    cid = lax.axis_index("c")
    sid = lax.axis_index("s")
    wid = sid * NC + cid
    ebase = wid * EPW

    def wait_scatters():
        for k in range(UNROLL):
            pltpu.make_async_copy(rows[k], acc.at[didxs[k]], ssems[k]).wait()

    def body(i, carry):
        @pl.when(i > 0)
        def _():
            wait_scatters()

        base = ebase + i * (UNROLL * CHUNK)
        loads = []
        for k in range(UNROLL):
            off = base + k * CHUNK
            sld = pltpu.async_copy(src_hbm.at[pl.ds(off, CHUNK)],
                                   sidxs[k], qsems[k])
            dld = pltpu.async_copy(dst_hbm.at[pl.ds(off, CHUNK)],
                                   didxs[k], dsems[k])
            loads.append((sld, dld))
        gathers = []
        for k in range(UNROLL):
            loads[k][0].wait()
            g = pltpu.async_copy(x_hbm.at[sidxs[k]], rows[k], gsems[k])
            gathers.append(g)
        for k in range(UNROLL):
            loads[k][1].wait()
            gathers[k].wait()
            pltpu.make_async_copy(rows[k], acc.at[didxs[k]],
                                  ssems[k]).start(add=True)
        return carry

    pltpu.sync_copy(zero_hbm.at[pl.ds(sid * RPT, RPT)],
                    acc.at[pl.ds(sid * RPT, RPT)])

    @pl.when(sid == NS - 1)
    def _():
        pltpu.sync_copy(zero_hbm.at[pl.ds(NS * RPT, TAIL)],
                        acc.at[pl.ds(NS * RPT, TAIL)])

    plsc.subcore_barrier()

    lax.fori_loop(0, NITER, body, 0)
    wait_scatters()
    toff = ebase + NFULL * CHUNK
    pltpu.sync_copy(src_hbm.at[pl.ds(toff, TAILE)], tsidx)
    pltpu.sync_copy(dst_hbm.at[pl.ds(toff, TAILE)], tdidx)
    pltpu.async_copy(x_hbm.at[tsidx], rows[0].at[pl.ds(0, TAILE)],
                     tsem).wait()
    pltpu.sync_copy(rows[0].at[pl.ds(0, TAILE)], acc.at[tdidx], add=True)
    plsc.subcore_barrier()
    obase = cid * N + sid * RPT
    pltpu.sync_copy(acc.at[pl.ds(sid * RPT, RPT)],
                    out_hbm.at[pl.ds(obase, RPT)])

    @pl.when(sid == NS - 1)
    def _():
        pltpu.sync_copy(acc.at[pl.ds(NS * RPT, TAIL)],
                        out_hbm.at[pl.ds(cid * N + NS * RPT, TAIL)])


@functools.lru_cache(maxsize=None)
def _edge_agg_kernel():
    return pl.kernel(
        _edge_agg_body,
        out_type=jax.ShapeDtypeStruct((NC * N, HP), jnp.float32),
        mesh=plsc.VectorSubcoreMesh(core_axis_name="c", subcore_axis_name="s",
                                    num_cores=NC, num_subcores=NS),
        scratch_types=[
            [pltpu.VMEM((CHUNK,), jnp.int32) for _ in range(UNROLL)],
            [pltpu.VMEM((CHUNK,), jnp.int32) for _ in range(UNROLL)],
            pltpu.VMEM((TAILE,), jnp.int32),
            pltpu.VMEM((TAILE,), jnp.int32),
            [pltpu.VMEM((CHUNK, HP), jnp.float32) for _ in range(UNROLL)],
            pltpu.VMEM_SHARED((N, HP), jnp.float32),
            [pltpu.SemaphoreType.DMA for _ in range(UNROLL)],
            [pltpu.SemaphoreType.DMA for _ in range(UNROLL)],
            [pltpu.SemaphoreType.DMA for _ in range(UNROLL)],
            [pltpu.SemaphoreType.DMA for _ in range(UNROLL)],
            pltpu.SemaphoreType.DMA,
        ],
    )


def _edge_agg(x0, src, dst, zeros):
    return _edge_agg_kernel()(x0, src, dst, zeros)


def kernel(x, edge_index, batch, fh_W1, fh_b1, fh_g1, fh_be1, fh_W2, fh_b2,
           fh_g2, fh_be2, c1_W1, c1_b1, c1_g1, c1_be1, c1_W2, c1_b2, c1_g2,
           c1_be2, c2_W1, c2_b1, c2_g1, c2_be1, c2_W2, c2_b2, c2_g2, c2_be2,
           lin_W0, lin_b0, lin_W1, lin_b1, lin_W2, lin_b2):
    src = edge_index[0]
    dst = edge_index[1]
    brow = batch.reshape(1, N)
    zeros = jnp.zeros((N, HP), jnp.float32)
    r = lambda a: a.reshape(1, -1)

    x0, out = _mlp_first(x, brow, fh_W1, r(fh_b1), r(fh_g1), r(fh_be1),
                         fh_W2, r(fh_b2), r(fh_g2), r(fh_be2),
                         lin_W0, r(lin_b0))
    agg = _edge_agg(x0, src, dst, zeros)
    x1, out = _mlp_layer(x0, agg, brow, c1_W1, r(c1_b1), r(c1_g1), r(c1_be1),
                         c1_W2, r(c1_b2), r(c1_g2), r(c1_be2),
                         lin_W1, r(lin_b1), out)
    agg = _edge_agg(x1, src, dst, zeros)
    _, out = _mlp_layer(x1, agg, brow, c2_W1, r(c2_b1), r(c2_g1), r(c2_be1),
                        c2_W2, r(c2_b2), r(c2_g2), r(c2_be2),
                        lin_W2, r(lin_b2), out)
    return out


# speedup vs baseline: 8.0274x; 1.0068x over previous
import functools

import jax
import jax.numpy as jnp
from jax import lax
from jax.experimental import pallas as pl
from jax.experimental.pallas import tpu as pltpu
from jax.experimental.pallas import tpu_sc as plsc

N = 10000
E = 320000
DF = 128
H = 64
T = 10
G = 64
HP = 128

NC = 2
NS = 16
NW = NC * NS
EPW = E // NW
CHUNK = 128
NFULL = EPW // CHUNK
TAILE = EPW - NFULL * CHUNK
RPT = 624
TAIL = N - NS * RPT


def _bn_relu(h, g, b):
    m = jnp.mean(h, axis=0, keepdims=True)
    v = jnp.mean((h - m) * (h - m), axis=0, keepdims=True)
    return jax.nn.relu((h - m) / jnp.sqrt(v + 1e-5) * g + b)


def _mlp(h, w1, b1, g1, be1, w2, b2, g2, be2):
    h = jnp.dot(h, w1, preferred_element_type=jnp.float32) + b1
    h = _bn_relu(h, g1, be1)
    h = jnp.dot(h, w2, preferred_element_type=jnp.float32) + b2
    h = _bn_relu(h, g2, be2)
    return h


def _pool(h, batch_row):
    onehot = (lax.broadcasted_iota(jnp.int32, (G, N), 0) == batch_row)
    onehot = onehot.astype(jnp.float32)
    pool = jnp.dot(onehot, h, preferred_element_type=jnp.float32)
    cnt = jnp.sum(onehot, axis=1, keepdims=True)
    return pool, cnt


def _first_body(x_ref, w1_ref, b1_ref, g1_ref, be1_ref,
                w2_ref, b2_ref, g2_ref, be2_ref, h_ref):
    h = _mlp(x_ref[...], w1_ref[...], b1_ref[...], g1_ref[...], be1_ref[...],
             w2_ref[...], b2_ref[...], g2_ref[...], be2_ref[...])
    h_ref[...] = jnp.concatenate([h, jnp.zeros_like(h)], axis=1)


def _layer_body(x_ref, agg_ref, w1_ref, b1_ref, g1_ref, be1_ref,
                w2_ref, b2_ref, g2_ref, be2_ref, h_ref):
    a = agg_ref[...][:, :H]
    xin = x_ref[...][:, :H] + a[:N] + a[N:]
    h = _mlp(xin, w1_ref[...], b1_ref[...], g1_ref[...], be1_ref[...],
             w2_ref[...], b2_ref[...], g2_ref[...], be2_ref[...])
    h_ref[...] = jnp.concatenate([h, jnp.zeros_like(h)], axis=1)


def _last_body(x_ref, agg_ref, b_ref, w1_ref, b1_ref, g1_ref, be1_ref,
               w2_ref, b2_ref, g2_ref, be2_ref, lw_ref, lb_ref, oin_ref,
               out_ref):
    a = agg_ref[...][:, :H]
    xin = x_ref[...][:, :H] + a[:N] + a[N:]
    h = _mlp(xin, w1_ref[...], b1_ref[...], g1_ref[...], be1_ref[...],
             w2_ref[...], b2_ref[...], g2_ref[...], be2_ref[...])
    pool, _ = _pool(h, b_ref[...])
    out_ref[...] = (oin_ref[...]
                    + jnp.dot(pool, lw_ref[...], preferred_element_type=jnp.float32)
                    + lb_ref[...])


def _pool_first_body(h_ref, b_ref, lw_ref, lb_ref, out_ref):
    pool, cnt = _pool(h_ref[...][:, :H], b_ref[...])
    out_ref[...] = (jnp.dot(pool, lw_ref[...], preferred_element_type=jnp.float32)
                    + cnt * lb_ref[...])


def _pool_body(h_ref, b_ref, lw_ref, lb_ref, oin_ref, out_ref):
    pool, _ = _pool(h_ref[...][:, :H], b_ref[...])
    out_ref[...] = (oin_ref[...]
                    + jnp.dot(pool, lw_ref[...], preferred_element_type=jnp.float32)
                    + lb_ref[...])


_mlp_first = pl.pallas_call(
    _first_body,
    out_shape=jax.ShapeDtypeStruct((N, HP), jnp.float32),
)

_mlp_layer = pl.pallas_call(
    _layer_body,
    out_shape=jax.ShapeDtypeStruct((N, HP), jnp.float32),
)

_mlp_last = pl.pallas_call(
    _last_body,
    out_shape=jax.ShapeDtypeStruct((G, T), jnp.float32),
)

_pool_first = pl.pallas_call(
    _pool_first_body,
    out_shape=jax.ShapeDtypeStruct((G, T), jnp.float32),
)

_pool_next = pl.pallas_call(
    _pool_body,
    out_shape=jax.ShapeDtypeStruct((G, T), jnp.float32),
)


UNROLL = 3
NITER = NFULL // UNROLL


def _edge_agg_body(x_hbm, src_hbm, dst_hbm, zero_hbm, out_hbm,
                   sidxs, didxs, tsidx, tdidx, rows, acc,
                   gsems, ssems, dsems, qsems, tsem):
    cid = lax.axis_index("c")
    sid = lax.axis_index("s")
    wid = sid * NC + cid
    ebase = wid * EPW

    def wait_scatters():
        for k in range(UNROLL):
            pltpu.make_async_copy(rows[k], acc.at[didxs[k]], ssems[k]).wait()

    def body(i, carry):
        @pl.when(i > 0)
        def _():
            wait_scatters()

        base = ebase + i * (UNROLL * CHUNK)
        loads = []
        for k in range(UNROLL):
            off = base + k * CHUNK
            sld = pltpu.async_copy(src_hbm.at[pl.ds(off, CHUNK)],
                                   sidxs[k], qsems[k])
            dld = pltpu.async_copy(dst_hbm.at[pl.ds(off, CHUNK)],
                                   didxs[k], dsems[k])
            loads.append((sld, dld))
        gathers = []
        for k in range(UNROLL):
            loads[k][0].wait()
            g = pltpu.async_copy(x_hbm.at[sidxs[k]], rows[k], gsems[k])
            gathers.append(g)
        for k in range(UNROLL):
            loads[k][1].wait()
            gathers[k].wait()
            pltpu.make_async_copy(rows[k], acc.at[didxs[k]],
                                  ssems[k]).start(add=True)
        return carry

    pltpu.sync_copy(zero_hbm.at[pl.ds(sid * RPT, RPT)],
                    acc.at[pl.ds(sid * RPT, RPT)])

    @pl.when(sid == NS - 1)
    def _():
        pltpu.sync_copy(zero_hbm.at[pl.ds(NS * RPT, TAIL)],
                        acc.at[pl.ds(NS * RPT, TAIL)])

    plsc.subcore_barrier()

    lax.fori_loop(0, NITER, body, 0)
    wait_scatters()
    toff = ebase + NFULL * CHUNK
    pltpu.sync_copy(src_hbm.at[pl.ds(toff, TAILE)], tsidx)
    pltpu.sync_copy(dst_hbm.at[pl.ds(toff, TAILE)], tdidx)
    pltpu.async_copy(x_hbm.at[tsidx], rows[0].at[pl.ds(0, TAILE)],
                     tsem).wait()
    pltpu.sync_copy(rows[0].at[pl.ds(0, TAILE)], acc.at[tdidx], add=True)
    plsc.subcore_barrier()
    obase = cid * N + sid * RPT
    pltpu.sync_copy(acc.at[pl.ds(sid * RPT, RPT)],
                    out_hbm.at[pl.ds(obase, RPT)])

    @pl.when(sid == NS - 1)
    def _():
        pltpu.sync_copy(acc.at[pl.ds(NS * RPT, TAIL)],
                        out_hbm.at[pl.ds(cid * N + NS * RPT, TAIL)])


@functools.lru_cache(maxsize=None)
def _edge_agg_kernel():
    return pl.kernel(
        _edge_agg_body,
        out_type=jax.ShapeDtypeStruct((NC * N, HP), jnp.float32),
        mesh=plsc.VectorSubcoreMesh(core_axis_name="c", subcore_axis_name="s",
                                    num_cores=NC, num_subcores=NS),
        scratch_types=[
            [pltpu.VMEM((CHUNK,), jnp.int32) for _ in range(UNROLL)],
            [pltpu.VMEM((CHUNK,), jnp.int32) for _ in range(UNROLL)],
            pltpu.VMEM((TAILE,), jnp.int32),
            pltpu.VMEM((TAILE,), jnp.int32),
            [pltpu.VMEM((CHUNK, HP), jnp.float32) for _ in range(UNROLL)],
            pltpu.VMEM_SHARED((N, HP), jnp.float32),
            [pltpu.SemaphoreType.DMA for _ in range(UNROLL)],
            [pltpu.SemaphoreType.DMA for _ in range(UNROLL)],
            [pltpu.SemaphoreType.DMA for _ in range(UNROLL)],
            [pltpu.SemaphoreType.DMA for _ in range(UNROLL)],
            pltpu.SemaphoreType.DMA,
        ],
    )


def _edge_agg(x0, src, dst, zeros):
    return _edge_agg_kernel()(x0, src, dst, zeros)


def kernel(x, edge_index, batch, fh_W1, fh_b1, fh_g1, fh_be1, fh_W2, fh_b2,
           fh_g2, fh_be2, c1_W1, c1_b1, c1_g1, c1_be1, c1_W2, c1_b2, c1_g2,
           c1_be2, c2_W1, c2_b1, c2_g1, c2_be1, c2_W2, c2_b2, c2_g2, c2_be2,
           lin_W0, lin_b0, lin_W1, lin_b1, lin_W2, lin_b2):
    src = edge_index[0]
    dst = edge_index[1]
    brow = batch.reshape(1, N)
    zeros = jnp.zeros((N, HP), jnp.float32)
    r = lambda a: a.reshape(1, -1)

    x0 = _mlp_first(x, fh_W1, r(fh_b1), r(fh_g1), r(fh_be1),
                    fh_W2, r(fh_b2), r(fh_g2), r(fh_be2))
    agg = _edge_agg(x0, src, dst, zeros)
    out = _pool_first(x0, brow, lin_W0, r(lin_b0))
    x1 = _mlp_layer(x0, agg, c1_W1, r(c1_b1), r(c1_g1), r(c1_be1),
                    c1_W2, r(c1_b2), r(c1_g2), r(c1_be2))
    agg = _edge_agg(x1, src, dst, zeros)
    out = _pool_next(x1, brow, lin_W1, r(lin_b1), out)
    out = _mlp_last(x1, agg, brow, c2_W1, r(c2_b1), r(c2_g1), r(c2_be1),
                    c2_W2, r(c2_b2), r(c2_g2), r(c2_be2),
                    lin_W2, r(lin_b2), out)
    return out


# speedup vs baseline: 10.6370x; 1.3251x over previous
import functools

import jax
import jax.numpy as jnp
from jax import lax
from jax.experimental import pallas as pl
from jax.experimental.pallas import tpu as pltpu
from jax.experimental.pallas import tpu_sc as plsc

N = 10000
E = 320000
DF = 128
H = 64
T = 10
G = 64
HP = 128

NC = 2
NS = 16
NW = NC * NS
EPW = E // NW
CHUNK = 64
GPC = 3
NBUF = 2 * GPC
NFULL = EPW // CHUNK
NGRP = NFULL // GPC
TAILE = EPW - NFULL * CHUNK
RPT = 624
TAIL = N - NS * RPT


def _bn_relu(h, g, b):
    m = jnp.mean(h, axis=0, keepdims=True)
    v = jnp.mean((h - m) * (h - m), axis=0, keepdims=True)
    return jax.nn.relu((h - m) / jnp.sqrt(v + 1e-5) * g + b)


def _mlp(h, w1, b1, g1, be1, w2, b2, g2, be2):
    h = jnp.dot(h, w1, preferred_element_type=jnp.float32) + b1
    h = _bn_relu(h, g1, be1)
    h = jnp.dot(h, w2, preferred_element_type=jnp.float32) + b2
    h = _bn_relu(h, g2, be2)
    return h


def _pool(h, batch_row):
    onehot = (lax.broadcasted_iota(jnp.int32, (G, N), 0) == batch_row)
    onehot = onehot.astype(jnp.float32)
    pool = jnp.dot(onehot, h, preferred_element_type=jnp.float32)
    cnt = jnp.sum(onehot, axis=1, keepdims=True)
    return pool, cnt


def _first_body(x_ref, w1_ref, b1_ref, g1_ref, be1_ref,
                w2_ref, b2_ref, g2_ref, be2_ref, h_ref):
    h = _mlp(x_ref[...], w1_ref[...], b1_ref[...], g1_ref[...], be1_ref[...],
             w2_ref[...], b2_ref[...], g2_ref[...], be2_ref[...])
    h_ref[...] = jnp.concatenate([h, jnp.zeros_like(h)], axis=1)


def _layer_body(x_ref, agg_ref, w1_ref, b1_ref, g1_ref, be1_ref,
                w2_ref, b2_ref, g2_ref, be2_ref, h_ref):
    a = agg_ref[...][:, :H]
    xin = x_ref[...][:, :H] + a[:N] + a[N:]
    h = _mlp(xin, w1_ref[...], b1_ref[...], g1_ref[...], be1_ref[...],
             w2_ref[...], b2_ref[...], g2_ref[...], be2_ref[...])
    h_ref[...] = jnp.concatenate([h, jnp.zeros_like(h)], axis=1)


def _last_body(x_ref, agg_ref, b_ref, w1_ref, b1_ref, g1_ref, be1_ref,
               w2_ref, b2_ref, g2_ref, be2_ref, lw_ref, lb_ref, oin_ref,
               out_ref):
    a = agg_ref[...][:, :H]
    xin = x_ref[...][:, :H] + a[:N] + a[N:]
    h = _mlp(xin, w1_ref[...], b1_ref[...], g1_ref[...], be1_ref[...],
             w2_ref[...], b2_ref[...], g2_ref[...], be2_ref[...])
    pool, _ = _pool(h, b_ref[...])
    out_ref[...] = (oin_ref[...]
                    + jnp.dot(pool, lw_ref[...], preferred_element_type=jnp.float32)
                    + lb_ref[...])


def _pool_first_body(h_ref, b_ref, lw_ref, lb_ref, out_ref):
    pool, cnt = _pool(h_ref[...][:, :H], b_ref[...])
    out_ref[...] = (jnp.dot(pool, lw_ref[...], preferred_element_type=jnp.float32)
                    + cnt * lb_ref[...])


def _pool_body(h_ref, b_ref, lw_ref, lb_ref, oin_ref, out_ref):
    pool, _ = _pool(h_ref[...][:, :H], b_ref[...])
    out_ref[...] = (oin_ref[...]
                    + jnp.dot(pool, lw_ref[...], preferred_element_type=jnp.float32)
                    + lb_ref[...])


_mlp_first = pl.pallas_call(
    _first_body,
    out_shape=jax.ShapeDtypeStruct((N, HP), jnp.float32),
)

_mlp_layer = pl.pallas_call(
    _layer_body,
    out_shape=jax.ShapeDtypeStruct((N, HP), jnp.float32),
)

_mlp_last = pl.pallas_call(
    _last_body,
    out_shape=jax.ShapeDtypeStruct((G, T), jnp.float32),
)

_pool_first = pl.pallas_call(
    _pool_first_body,
    out_shape=jax.ShapeDtypeStruct((G, T), jnp.float32),
)

_pool_next = pl.pallas_call(
    _pool_body,
    out_shape=jax.ShapeDtypeStruct((G, T), jnp.float32),
)


def _edge_agg_body(x_hbm, edge_hbm, zero_hbm, out_hbm,
                   sidxs, didxs, tsidx, tdidx, rows, acc,
                   gsems, ssems, dsems, qsems, tsem):
    cid = lax.axis_index("c")
    sid = lax.axis_index("s")
    wid = sid * NC + cid
    ebase = wid * EPW
    BUF_A = list(range(GPC))
    BUF_B = list(range(GPC, NBUF))

    def wait_scatter(k):
        pltpu.make_async_copy(rows[k], acc.at[didxs[k]], ssems[k]).wait()

    def wait_gather(k):
        pltpu.make_async_copy(x_hbm.at[sidxs[k]], rows[k], gsems[k]).wait()

    def wait_idx(ref, sem):
        pltpu.make_async_copy(edge_hbm.at[pl.ds(0, CHUNK)], ref, sem).wait()

    def fire_scatters(bufs):
        for k in bufs:
            wait_idx(didxs[k], dsems[k])
            wait_gather(k)
            pltpu.make_async_copy(rows[k], acc.at[didxs[k]],
                                  ssems[k]).start(add=True)

    def phase(gi, cur, prv):
        @pl.when(gi > 1)
        def _():
            for k in cur:
                wait_scatter(k)

        base = ebase + gi * (GPC * CHUNK)
        for j, k in enumerate(cur):
            off = base + j * CHUNK
            pltpu.async_copy(edge_hbm.at[pl.ds(off, CHUNK)],
                             sidxs[k], qsems[k])
            pltpu.async_copy(edge_hbm.at[pl.ds(E + off, CHUNK)],
                             didxs[k], dsems[k])
        for k in cur:
            wait_idx(sidxs[k], qsems[k])
            pltpu.make_async_copy(x_hbm.at[sidxs[k]], rows[k],
                                  gsems[k]).start()
        @pl.when(gi > 0)
        def _():
            fire_scatters(prv)

    def body(i, carry):
        @pl.when(i % 2 == 0)
        def _():
            phase(i, BUF_A, BUF_B)

        @pl.when(i % 2 == 1)
        def _():
            phase(i, BUF_B, BUF_A)
        return carry

    pltpu.sync_copy(zero_hbm.at[pl.ds(sid * RPT, RPT)],
                    acc.at[pl.ds(sid * RPT, RPT)])

    @pl.when(sid == NS - 1)
    def _():
        pltpu.sync_copy(zero_hbm.at[pl.ds(NS * RPT, TAIL)],
                        acc.at[pl.ds(NS * RPT, TAIL)])

    plsc.subcore_barrier()

    lax.fori_loop(0, NGRP, body, 0)
    fire_scatters(BUF_B)
    for k in range(NBUF):
        wait_scatter(k)
    toff = ebase + NFULL * CHUNK
    pltpu.sync_copy(edge_hbm.at[pl.ds(toff, TAILE)], tsidx)
    pltpu.sync_copy(edge_hbm.at[pl.ds(E + toff, TAILE)], tdidx)
    pltpu.async_copy(x_hbm.at[tsidx], rows[0].at[pl.ds(0, TAILE)],
                     tsem).wait()
    pltpu.sync_copy(rows[0].at[pl.ds(0, TAILE)], acc.at[tdidx], add=True)
    plsc.subcore_barrier()
    obase = cid * N + sid * RPT
    pltpu.sync_copy(acc.at[pl.ds(sid * RPT, RPT)],
                    out_hbm.at[pl.ds(obase, RPT)])

    @pl.when(sid == NS - 1)
    def _():
        pltpu.sync_copy(acc.at[pl.ds(NS * RPT, TAIL)],
                        out_hbm.at[pl.ds(cid * N + NS * RPT, TAIL)])


@functools.lru_cache(maxsize=None)
def _edge_agg_kernel():
    return pl.kernel(
        _edge_agg_body,
        out_type=jax.ShapeDtypeStruct((NC * N, HP), jnp.float32),
        mesh=plsc.VectorSubcoreMesh(core_axis_name="c", subcore_axis_name="s",
                                    num_cores=NC, num_subcores=NS),
        scratch_types=[
            [pltpu.VMEM((CHUNK,), jnp.int32) for _ in range(NBUF)],
            [pltpu.VMEM((CHUNK,), jnp.int32) for _ in range(NBUF)],
            pltpu.VMEM((TAILE,), jnp.int32),
            pltpu.VMEM((TAILE,), jnp.int32),
            [pltpu.VMEM((CHUNK, HP), jnp.float32) for _ in range(NBUF)],
            pltpu.VMEM_SHARED((N, HP), jnp.float32),
            [pltpu.SemaphoreType.DMA for _ in range(NBUF)],
            [pltpu.SemaphoreType.DMA for _ in range(NBUF)],
            [pltpu.SemaphoreType.DMA for _ in range(NBUF)],
            [pltpu.SemaphoreType.DMA for _ in range(NBUF)],
            pltpu.SemaphoreType.DMA,
        ],
    )


def _edge_agg(x0, eflat, zeros):
    return _edge_agg_kernel()(x0, eflat, zeros)


def kernel(x, edge_index, batch, fh_W1, fh_b1, fh_g1, fh_be1, fh_W2, fh_b2,
           fh_g2, fh_be2, c1_W1, c1_b1, c1_g1, c1_be1, c1_W2, c1_b2, c1_g2,
           c1_be2, c2_W1, c2_b1, c2_g1, c2_be1, c2_W2, c2_b2, c2_g2, c2_be2,
           lin_W0, lin_b0, lin_W1, lin_b1, lin_W2, lin_b2):
    eflat = edge_index.reshape(2 * E)
    brow = batch.reshape(1, N)
    zeros = jnp.zeros((N, HP), jnp.float32)
    r = lambda a: a.reshape(1, -1)

    x0 = _mlp_first(x, fh_W1, r(fh_b1), r(fh_g1), r(fh_be1),
                    fh_W2, r(fh_b2), r(fh_g2), r(fh_be2))
    agg = _edge_agg(x0, eflat, zeros)
    out = _pool_first(x0, brow, lin_W0, r(lin_b0))
    x1 = _mlp_layer(x0, agg, c1_W1, r(c1_b1), r(c1_g1), r(c1_be1),
                    c1_W2, r(c1_b2), r(c1_g2), r(c1_be2))
    agg = _edge_agg(x1, eflat, zeros)
    out = _pool_next(x1, brow, lin_W1, r(lin_b1), out)
    out = _mlp_last(x1, agg, brow, c2_W1, r(c2_b1), r(c2_g1), r(c2_be1),
                    c2_W2, r(c2_b2), r(c2_g2), r(c2_be2),
                    lin_W2, r(lin_b2), out)
    return out


# speedup vs baseline: 10.8446x; 1.0195x over previous
import functools

import jax
import jax.numpy as jnp
from jax import lax
from jax.experimental import pallas as pl
from jax.experimental.pallas import tpu as pltpu
from jax.experimental.pallas import tpu_sc as plsc

N = 10000
E = 320000
DF = 128
H = 64
T = 10
G = 64
HP = 128

NC = 2
NS = 16
NW = NC * NS
EPW = E // NW
CHUNK = 64
GPC = 3
NBUF = 2 * GPC
NFULL = EPW // CHUNK
NGRP = NFULL // GPC
TAILE = EPW - NFULL * CHUNK
RPT = 624
TAIL = N - NS * RPT


def _bn_relu(h, g, b):
    m = jnp.mean(h, axis=0, keepdims=True)
    m2 = jnp.mean(h * h, axis=0, keepdims=True)
    v = m2 - m * m
    return jax.nn.relu((h - m) / jnp.sqrt(v + 1e-5) * g + b)


def _mlp(h, w1, b1, g1, be1, w2, b2, g2, be2):
    h = jnp.dot(h, w1, preferred_element_type=jnp.float32) + b1
    h = _bn_relu(h, g1, be1)
    h = jnp.dot(h, w2, preferred_element_type=jnp.float32) + b2
    h = _bn_relu(h, g2, be2)
    return h


def _pool(h, batch_row):
    onehot = (lax.broadcasted_iota(jnp.int32, (G, N), 0) == batch_row)
    onehot = onehot.astype(jnp.float32)
    pool = jnp.dot(onehot, h, preferred_element_type=jnp.float32)
    cnt = jnp.sum(onehot, axis=1, keepdims=True)
    return pool, cnt


def _first_body(x_ref, w1_ref, b1_ref, g1_ref, be1_ref,
                w2_ref, b2_ref, g2_ref, be2_ref, h_ref):
    h = _mlp(x_ref[...], w1_ref[...], b1_ref[...], g1_ref[...], be1_ref[...],
             w2_ref[...], b2_ref[...], g2_ref[...], be2_ref[...])
    h_ref[...] = jnp.concatenate([h, jnp.zeros_like(h)], axis=1)


def _layer_body(x_ref, agg_ref, w1_ref, b1_ref, g1_ref, be1_ref,
                w2_ref, b2_ref, g2_ref, be2_ref, h_ref):
    a = agg_ref[...][:, :H]
    xin = x_ref[...][:, :H] + a[:N] + a[N:]
    h = _mlp(xin, w1_ref[...], b1_ref[...], g1_ref[...], be1_ref[...],
             w2_ref[...], b2_ref[...], g2_ref[...], be2_ref[...])
    h_ref[...] = jnp.concatenate([h, jnp.zeros_like(h)], axis=1)


def _last_body(x_ref, agg_ref, b_ref, w1_ref, b1_ref, g1_ref, be1_ref,
               w2_ref, b2_ref, g2_ref, be2_ref, lw_ref, lb_ref, oin_ref,
               out_ref):
    a = agg_ref[...][:, :H]
    xin = x_ref[...][:, :H] + a[:N] + a[N:]
    h = _mlp(xin, w1_ref[...], b1_ref[...], g1_ref[...], be1_ref[...],
             w2_ref[...], b2_ref[...], g2_ref[...], be2_ref[...])
    pool, _ = _pool(h, b_ref[...])
    out_ref[...] = (oin_ref[...]
                    + jnp.dot(pool, lw_ref[...], preferred_element_type=jnp.float32)
                    + lb_ref[...])


def _pool_first_body(h_ref, b_ref, lw_ref, lb_ref, out_ref):
    pool, cnt = _pool(h_ref[...][:, :H], b_ref[...])
    out_ref[...] = (jnp.dot(pool, lw_ref[...], preferred_element_type=jnp.float32)
                    + cnt * lb_ref[...])


def _pool_body(h_ref, b_ref, lw_ref, lb_ref, oin_ref, out_ref):
    pool, _ = _pool(h_ref[...][:, :H], b_ref[...])
    out_ref[...] = (oin_ref[...]
                    + jnp.dot(pool, lw_ref[...], preferred_element_type=jnp.float32)
                    + lb_ref[...])


def _full(*s):
    return pl.BlockSpec(s, lambda: (0,) * len(s))


_xhalf = _full(N, HP)
_agghalf = _full(NC * N, HP)
_w = _full(H, H)
_v = _full(1, H)
_lw = _full(H, T)
_lb = _full(1, T)
_bn = _full(1, N)
_ot = _full(G, T)

_mlp_first = pl.pallas_call(
    _first_body,
    in_specs=[_full(N, DF), _full(DF, H), _v, _v, _v, _w, _v, _v, _v],
    out_specs=_full(N, HP),
    out_shape=jax.ShapeDtypeStruct((N, HP), jnp.float32),
)

_mlp_layer = pl.pallas_call(
    _layer_body,
    in_specs=[_xhalf, _agghalf, _w, _v, _v, _v, _w, _v, _v, _v],
    out_specs=_full(N, HP),
    out_shape=jax.ShapeDtypeStruct((N, HP), jnp.float32),
)

_mlp_last = pl.pallas_call(
    _last_body,
    in_specs=[_xhalf, _agghalf, _bn, _w, _v, _v, _v, _w, _v, _v, _v,
              _lw, _lb, _ot],
    out_specs=_ot,
    out_shape=jax.ShapeDtypeStruct((G, T), jnp.float32),
)

_pool_first = pl.pallas_call(
    _pool_first_body,
    in_specs=[_xhalf, _bn, _lw, _lb],
    out_specs=_ot,
    out_shape=jax.ShapeDtypeStruct((G, T), jnp.float32),
)

_pool_next = pl.pallas_call(
    _pool_body,
    in_specs=[_xhalf, _bn, _lw, _lb, _ot],
    out_specs=_ot,
    out_shape=jax.ShapeDtypeStruct((G, T), jnp.float32),
)


def _edge_agg_body(x_hbm, edge_hbm, zero_hbm, out_hbm,
                   sidxs, didxs, tsidx, tdidx, rows, acc,
                   gsems, ssems, dsems, qsems, tsem):
    cid = lax.axis_index("c")
    sid = lax.axis_index("s")
    wid = sid * NC + cid
    ebase = wid * EPW
    BUF_A = list(range(GPC))
    BUF_B = list(range(GPC, NBUF))

    def wait_scatter(k):
        pltpu.make_async_copy(rows[k], acc.at[didxs[k]], ssems[k]).wait()

    def wait_gather(k):
        pltpu.make_async_copy(x_hbm.at[sidxs[k]], rows[k], gsems[k]).wait()

    def wait_idx(ref, sem):
        pltpu.make_async_copy(edge_hbm.at[pl.ds(0, CHUNK)], ref, sem).wait()

    def fire_scatters(bufs):
        for k in bufs:
            wait_idx(didxs[k], dsems[k])
            wait_gather(k)
            pltpu.make_async_copy(rows[k], acc.at[didxs[k]],
                                  ssems[k]).start(add=True)

    def phase(gi, cur, prv):
        @pl.when(gi > 1)
        def _():
            for k in cur:
                wait_scatter(k)

        base = ebase + gi * (GPC * CHUNK)
        for j, k in enumerate(cur):
            off = base + j * CHUNK
            pltpu.async_copy(edge_hbm.at[pl.ds(off, CHUNK)],
                             sidxs[k], qsems[k])
            pltpu.async_copy(edge_hbm.at[pl.ds(E + off, CHUNK)],
                             didxs[k], dsems[k])
        for k in cur:
            wait_idx(sidxs[k], qsems[k])
            pltpu.make_async_copy(x_hbm.at[sidxs[k]], rows[k],
                                  gsems[k]).start()
        @pl.when(gi > 0)
        def _():
            fire_scatters(prv)

    def body(i, carry):
        @pl.when(i % 2 == 0)
        def _():
            phase(i, BUF_A, BUF_B)

        @pl.when(i % 2 == 1)
        def _():
            phase(i, BUF_B, BUF_A)
        return carry

    pltpu.sync_copy(zero_hbm.at[pl.ds(sid * RPT, RPT)],
                    acc.at[pl.ds(sid * RPT, RPT)])

    @pl.when(sid == NS - 1)
    def _():
        pltpu.sync_copy(zero_hbm.at[pl.ds(NS * RPT, TAIL)],
                        acc.at[pl.ds(NS * RPT, TAIL)])

    plsc.subcore_barrier()

    lax.fori_loop(0, NGRP, body, 0)
    fire_scatters(BUF_B)
    for k in range(NBUF):
        wait_scatter(k)
    toff = ebase + NFULL * CHUNK
    pltpu.sync_copy(edge_hbm.at[pl.ds(toff, TAILE)], tsidx)
    pltpu.sync_copy(edge_hbm.at[pl.ds(E + toff, TAILE)], tdidx)
    pltpu.async_copy(x_hbm.at[tsidx], rows[0].at[pl.ds(0, TAILE)],
                     tsem).wait()
    pltpu.sync_copy(rows[0].at[pl.ds(0, TAILE)], acc.at[tdidx], add=True)
    plsc.subcore_barrier()
    obase = cid * N + sid * RPT
    pltpu.sync_copy(acc.at[pl.ds(sid * RPT, RPT)],
                    out_hbm.at[pl.ds(obase, RPT)])

    @pl.when(sid == NS - 1)
    def _():
        pltpu.sync_copy(acc.at[pl.ds(NS * RPT, TAIL)],
                        out_hbm.at[pl.ds(cid * N + NS * RPT, TAIL)])


@functools.lru_cache(maxsize=None)
def _edge_agg_kernel():
    return pl.kernel(
        _edge_agg_body,
        out_type=jax.ShapeDtypeStruct((NC * N, HP), jnp.float32),
        mesh=plsc.VectorSubcoreMesh(core_axis_name="c", subcore_axis_name="s",
                                    num_cores=NC, num_subcores=NS),
        scratch_types=[
            [pltpu.VMEM((CHUNK,), jnp.int32) for _ in range(NBUF)],
            [pltpu.VMEM((CHUNK,), jnp.int32) for _ in range(NBUF)],
            pltpu.VMEM((TAILE,), jnp.int32),
            pltpu.VMEM((TAILE,), jnp.int32),
            [pltpu.VMEM((CHUNK, HP), jnp.float32) for _ in range(NBUF)],
            pltpu.VMEM_SHARED((N, HP), jnp.float32),
            [pltpu.SemaphoreType.DMA for _ in range(NBUF)],
            [pltpu.SemaphoreType.DMA for _ in range(NBUF)],
            [pltpu.SemaphoreType.DMA for _ in range(NBUF)],
            [pltpu.SemaphoreType.DMA for _ in range(NBUF)],
            pltpu.SemaphoreType.DMA,
        ],
    )


def _edge_agg(x0, eflat, zeros):
    return _edge_agg_kernel()(x0, eflat, zeros)


def kernel(x, edge_index, batch, fh_W1, fh_b1, fh_g1, fh_be1, fh_W2, fh_b2,
           fh_g2, fh_be2, c1_W1, c1_b1, c1_g1, c1_be1, c1_W2, c1_b2, c1_g2,
           c1_be2, c2_W1, c2_b1, c2_g1, c2_be1, c2_W2, c2_b2, c2_g2, c2_be2,
           lin_W0, lin_b0, lin_W1, lin_b1, lin_W2, lin_b2):
    eflat = edge_index.reshape(2 * E)
    brow = batch.reshape(1, N)
    zeros = jnp.zeros((N, HP), jnp.float32)
    r = lambda a: a.reshape(1, -1)

    x0 = _mlp_first(x, fh_W1, r(fh_b1), r(fh_g1), r(fh_be1),
                    fh_W2, r(fh_b2), r(fh_g2), r(fh_be2))
    agg = _edge_agg(x0, eflat, zeros)
    out = _pool_first(x0, brow, lin_W0, r(lin_b0))
    x1 = _mlp_layer(x0, agg, c1_W1, r(c1_b1), r(c1_g1), r(c1_be1),
                    c1_W2, r(c1_b2), r(c1_g2), r(c1_be2))
    agg = _edge_agg(x1, eflat, zeros)
    out = _pool_next(x1, brow, lin_W1, r(lin_b1), out)
    out = _mlp_last(x1, agg, brow, c2_W1, r(c2_b1), r(c2_g1), r(c2_be1),
                    c2_W2, r(c2_b2), r(c2_g2), r(c2_be2),
                    lin_W2, r(lin_b2), out)
    return out


# speedup vs baseline: 10.9368x; 1.0085x over previous
import functools

import jax
import jax.numpy as jnp
from jax import lax
from jax.experimental import pallas as pl
from jax.experimental.pallas import tpu as pltpu
from jax.experimental.pallas import tpu_sc as plsc

N = 10000
E = 320000
DF = 128
H = 64
T = 10
G = 64
HP = 128

NC = 2
NS = 16
NW = NC * NS
EPW = E // NW
CHUNK = 64
GPC = 3
NBUF = 2 * GPC
NFULL = EPW // CHUNK
NGRP = NFULL // GPC
TAILE = EPW - NFULL * CHUNK
RPT = 624
TAIL = N - NS * RPT


def _bn_relu(h, g, b):
    m = jnp.mean(h, axis=0, keepdims=True)
    m2 = jnp.mean(h * h, axis=0, keepdims=True)
    v = m2 - m * m
    return jax.nn.relu((h - m) / jnp.sqrt(v + 1e-5) * g + b)


def _mlp(h, w1, b1, g1, be1, w2, b2, g2, be2):
    h = jnp.dot(h, w1, preferred_element_type=jnp.float32) + b1
    h = _bn_relu(h, g1, be1)
    h = jnp.dot(h, w2, preferred_element_type=jnp.float32) + b2
    h = _bn_relu(h, g2, be2)
    return h


def _pool(h, batch_row):
    onehot = (lax.broadcasted_iota(jnp.int32, (G, N), 0) == batch_row)
    onehot = onehot.astype(jnp.float32)
    pool = jnp.dot(onehot, h, preferred_element_type=jnp.float32)
    cnt = jnp.sum(onehot, axis=1, keepdims=True)
    return pool, cnt


def _first_body(x_ref, w1_ref, b1_ref, g1_ref, be1_ref,
                w2_ref, b2_ref, g2_ref, be2_ref, h_ref):
    h = _mlp(x_ref[...], w1_ref[...], b1_ref[...], g1_ref[...], be1_ref[...],
             w2_ref[...], b2_ref[...], g2_ref[...], be2_ref[...])
    h_ref[...] = jnp.concatenate([h, jnp.zeros_like(h)], axis=1)


def _layer_body(x_ref, agg_ref, w1_ref, b1_ref, g1_ref, be1_ref,
                w2_ref, b2_ref, g2_ref, be2_ref, h_ref):
    a = agg_ref[...][:, :H]
    xin = x_ref[...][:, :H] + a[:N] + a[N:]
    h = _mlp(xin, w1_ref[...], b1_ref[...], g1_ref[...], be1_ref[...],
             w2_ref[...], b2_ref[...], g2_ref[...], be2_ref[...])
    h_ref[...] = jnp.concatenate([h, jnp.zeros_like(h)], axis=1)


def _last_body(x_ref, agg_ref, b_ref, w1_ref, b1_ref, g1_ref, be1_ref,
               w2_ref, b2_ref, g2_ref, be2_ref, lw_ref, lb_ref, oin_ref,
               out_ref):
    a = agg_ref[...][:, :H]
    xin = x_ref[...][:, :H] + a[:N] + a[N:]
    h = _mlp(xin, w1_ref[...], b1_ref[...], g1_ref[...], be1_ref[...],
             w2_ref[...], b2_ref[...], g2_ref[...], be2_ref[...])
    pool, _ = _pool(h, b_ref[...])
    out_ref[...] = (oin_ref[...]
                    + jnp.dot(pool, lw_ref[...], preferred_element_type=jnp.float32)
                    + lb_ref[...])


def _pool_first_body(h_ref, b_ref, lw_ref, lb_ref, out_ref):
    pool, cnt = _pool(h_ref[...][:, :H], b_ref[...])
    out_ref[...] = (jnp.dot(pool, lw_ref[...], preferred_element_type=jnp.float32)
                    + cnt * lb_ref[...])


def _pool_body(h_ref, b_ref, lw_ref, lb_ref, oin_ref, out_ref):
    pool, _ = _pool(h_ref[...][:, :H], b_ref[...])
    out_ref[...] = (oin_ref[...]
                    + jnp.dot(pool, lw_ref[...], preferred_element_type=jnp.float32)
                    + lb_ref[...])


def _zeros_body(z_ref):
    z_ref[...] = jnp.zeros((N, HP), jnp.float32)


_make_zeros = pl.pallas_call(
    _zeros_body,
    out_shape=jax.ShapeDtypeStruct((N, HP), jnp.float32),
)


def _full(*s):
    return pl.BlockSpec(s, lambda: (0,) * len(s))


_xhalf = _full(N, HP)
_agghalf = _full(NC * N, HP)
_w = _full(H, H)
_v = _full(1, H)
_lw = _full(H, T)
_lb = _full(1, T)
_bn = _full(1, N)
_ot = _full(G, T)

_mlp_first = pl.pallas_call(
    _first_body,
    in_specs=[_full(N, DF), _full(DF, H), _v, _v, _v, _w, _v, _v, _v],
    out_specs=_full(N, HP),
    out_shape=jax.ShapeDtypeStruct((N, HP), jnp.float32),
)

_mlp_layer = pl.pallas_call(
    _layer_body,
    in_specs=[_xhalf, _agghalf, _w, _v, _v, _v, _w, _v, _v, _v],
    out_specs=_full(N, HP),
    out_shape=jax.ShapeDtypeStruct((N, HP), jnp.float32),
)

_mlp_last = pl.pallas_call(
    _last_body,
    in_specs=[_xhalf, _agghalf, _bn, _w, _v, _v, _v, _w, _v, _v, _v,
              _lw, _lb, _ot],
    out_specs=_ot,
    out_shape=jax.ShapeDtypeStruct((G, T), jnp.float32),
)

_pool_first = pl.pallas_call(
    _pool_first_body,
    in_specs=[_xhalf, _bn, _lw, _lb],
    out_specs=_ot,
    out_shape=jax.ShapeDtypeStruct((G, T), jnp.float32),
)

_pool_next = pl.pallas_call(
    _pool_body,
    in_specs=[_xhalf, _bn, _lw, _lb, _ot],
    out_specs=_ot,
    out_shape=jax.ShapeDtypeStruct((G, T), jnp.float32),
)


def _edge_agg_body(x_hbm, edge_hbm, zero_hbm, out_hbm,
                   sidxs, didxs, tsidx, tdidx, rows, acc,
                   gsems, ssems, dsems, qsems, tsem):
    cid = lax.axis_index("c")
    sid = lax.axis_index("s")
    wid = sid * NC + cid
    ebase = wid * EPW
    BUF_A = list(range(GPC))
    BUF_B = list(range(GPC, NBUF))

    def wait_scatter(k):
        pltpu.make_async_copy(rows[k], acc.at[didxs[k]], ssems[k]).wait()

    def wait_gather(k):
        pltpu.make_async_copy(x_hbm.at[sidxs[k]], rows[k], gsems[k]).wait()

    def wait_idx(ref, sem):
        pltpu.make_async_copy(edge_hbm.at[pl.ds(0, CHUNK)], ref, sem).wait()

    def fire_scatters(bufs):
        for k in bufs:
            wait_idx(didxs[k], dsems[k])
            wait_gather(k)
            pltpu.make_async_copy(rows[k], acc.at[didxs[k]],
                                  ssems[k]).start(add=True)

    def phase(gi, cur, prv):
        @pl.when(gi > 1)
        def _():
            for k in cur:
                wait_scatter(k)

        base = ebase + gi * (GPC * CHUNK)
        for j, k in enumerate(cur):
            off = base + j * CHUNK
            pltpu.async_copy(edge_hbm.at[pl.ds(off, CHUNK)],
                             sidxs[k], qsems[k])
            pltpu.async_copy(edge_hbm.at[pl.ds(E + off, CHUNK)],
                             didxs[k], dsems[k])
        for k in cur:
            wait_idx(sidxs[k], qsems[k])
            pltpu.make_async_copy(x_hbm.at[sidxs[k]], rows[k],
                                  gsems[k]).start()
        @pl.when(gi > 0)
        def _():
            fire_scatters(prv)

    def body(i, carry):
        @pl.when(i % 2 == 0)
        def _():
            phase(i, BUF_A, BUF_B)

        @pl.when(i % 2 == 1)
        def _():
            phase(i, BUF_B, BUF_A)
        return carry

    pltpu.sync_copy(zero_hbm.at[pl.ds(sid * RPT, RPT)],
                    acc.at[pl.ds(sid * RPT, RPT)])

    @pl.when(sid == NS - 1)
    def _():
        pltpu.sync_copy(zero_hbm.at[pl.ds(NS * RPT, TAIL)],
                        acc.at[pl.ds(NS * RPT, TAIL)])

    plsc.subcore_barrier()

    lax.fori_loop(0, NGRP, body, 0)
    fire_scatters(BUF_B)
    for k in range(NBUF):
        wait_scatter(k)
    toff = ebase + NFULL * CHUNK
    pltpu.sync_copy(edge_hbm.at[pl.ds(toff, TAILE)], tsidx)
    pltpu.sync_copy(edge_hbm.at[pl.ds(E + toff, TAILE)], tdidx)
    pltpu.async_copy(x_hbm.at[tsidx], rows[0].at[pl.ds(0, TAILE)],
                     tsem).wait()
    pltpu.sync_copy(rows[0].at[pl.ds(0, TAILE)], acc.at[tdidx], add=True)
    plsc.subcore_barrier()
    obase = cid * N + sid * RPT
    pltpu.sync_copy(acc.at[pl.ds(sid * RPT, RPT)],
                    out_hbm.at[pl.ds(obase, RPT)])

    @pl.when(sid == NS - 1)
    def _():
        pltpu.sync_copy(acc.at[pl.ds(NS * RPT, TAIL)],
                        out_hbm.at[pl.ds(cid * N + NS * RPT, TAIL)])


@functools.lru_cache(maxsize=None)
def _edge_agg_kernel():
    return pl.kernel(
        _edge_agg_body,
        out_type=jax.ShapeDtypeStruct((NC * N, HP), jnp.float32),
        mesh=plsc.VectorSubcoreMesh(core_axis_name="c", subcore_axis_name="s",
                                    num_cores=NC, num_subcores=NS),
        scratch_types=[
            [pltpu.VMEM((CHUNK,), jnp.int32) for _ in range(NBUF)],
            [pltpu.VMEM((CHUNK,), jnp.int32) for _ in range(NBUF)],
            pltpu.VMEM((TAILE,), jnp.int32),
            pltpu.VMEM((TAILE,), jnp.int32),
            [pltpu.VMEM((CHUNK, HP), jnp.float32) for _ in range(NBUF)],
            pltpu.VMEM_SHARED((N, HP), jnp.float32),
            [pltpu.SemaphoreType.DMA for _ in range(NBUF)],
            [pltpu.SemaphoreType.DMA for _ in range(NBUF)],
            [pltpu.SemaphoreType.DMA for _ in range(NBUF)],
            [pltpu.SemaphoreType.DMA for _ in range(NBUF)],
            pltpu.SemaphoreType.DMA,
        ],
    )


def _edge_agg(x0, eflat, zeros):
    return _edge_agg_kernel()(x0, eflat, zeros)


def kernel(x, edge_index, batch, fh_W1, fh_b1, fh_g1, fh_be1, fh_W2, fh_b2,
           fh_g2, fh_be2, c1_W1, c1_b1, c1_g1, c1_be1, c1_W2, c1_b2, c1_g2,
           c1_be2, c2_W1, c2_b1, c2_g1, c2_be1, c2_W2, c2_b2, c2_g2, c2_be2,
           lin_W0, lin_b0, lin_W1, lin_b1, lin_W2, lin_b2):
    eflat = edge_index.reshape(2 * E)
    brow = batch.reshape(1, N)
    zeros = _make_zeros()
    r = lambda a: a.reshape(1, -1)

    x0 = _mlp_first(x, fh_W1, r(fh_b1), r(fh_g1), r(fh_be1),
                    fh_W2, r(fh_b2), r(fh_g2), r(fh_be2))
    agg = _edge_agg(x0, eflat, zeros)
    out = _pool_first(x0, brow, lin_W0, r(lin_b0))
    x1 = _mlp_layer(x0, agg, c1_W1, r(c1_b1), r(c1_g1), r(c1_be1),
                    c1_W2, r(c1_b2), r(c1_g2), r(c1_be2))
    agg = _edge_agg(x1, eflat, zeros)
    out = _pool_next(x1, brow, lin_W1, r(lin_b1), out)
    out = _mlp_last(x1, agg, brow, c2_W1, r(c2_b1), r(c2_g1), r(c2_be1),
                    c2_W2, r(c2_b2), r(c2_g2), r(c2_be2),
                    lin_W2, r(lin_b2), out)
    return out


# speedup vs baseline: 11.0142x; 1.0071x over previous
import functools

import jax
import jax.numpy as jnp
from jax import lax
from jax.experimental import pallas as pl
from jax.experimental.pallas import tpu as pltpu
from jax.experimental.pallas import tpu_sc as plsc

N = 10000
E = 320000
DF = 128
H = 64
T = 10
G = 64
HP = 128

NC = 2
NS = 16
NW = NC * NS
EPW = E // NW
CHUNK = 64
GPC = 3
NBUF = 2 * GPC
NFULL = EPW // CHUNK
NGRP = NFULL // GPC
TAILE = EPW - NFULL * CHUNK
RPT = 624
TAIL = N - NS * RPT


def _bn_relu(h, g, b):
    m = jnp.mean(h, axis=0, keepdims=True)
    m2 = jnp.mean(h * h, axis=0, keepdims=True)
    v = m2 - m * m
    return jax.nn.relu((h - m) / jnp.sqrt(v + 1e-5) * g + b)


def _mlp(h, w1, b1, g1, be1, w2, b2, g2, be2):
    h = jnp.dot(h, w1, preferred_element_type=jnp.float32) + b1
    h = _bn_relu(h, g1, be1)
    h = jnp.dot(h, w2, preferred_element_type=jnp.float32) + b2
    h = _bn_relu(h, g2, be2)
    return h


def _pool(h, batch_row):
    onehot = (lax.broadcasted_iota(jnp.int32, (G, N), 0) == batch_row)
    onehot = onehot.astype(jnp.float32)
    pool = jnp.dot(onehot, h, preferred_element_type=jnp.float32)
    cnt = jnp.sum(onehot, axis=1, keepdims=True)
    return pool, cnt


def _first_body(x_ref, w1_ref, b1_ref, g1_ref, be1_ref,
                w2_ref, b2_ref, g2_ref, be2_ref, h_ref):
    h = _mlp(x_ref[...], w1_ref[...], b1_ref[...], g1_ref[...], be1_ref[...],
             w2_ref[...], b2_ref[...], g2_ref[...], be2_ref[...])
    h_ref[...] = jnp.concatenate([h, jnp.zeros_like(h)], axis=1)


def _layer_body(x_ref, agg_ref, w1_ref, b1_ref, g1_ref, be1_ref,
                w2_ref, b2_ref, g2_ref, be2_ref, h_ref):
    a = agg_ref[...][:, :H]
    xin = x_ref[...][:, :H] + a[:N] + a[N:]
    h = _mlp(xin, w1_ref[...], b1_ref[...], g1_ref[...], be1_ref[...],
             w2_ref[...], b2_ref[...], g2_ref[...], be2_ref[...])
    h_ref[...] = jnp.concatenate([h, jnp.zeros_like(h)], axis=1)


def _last_body(x_ref, agg_ref, b_ref, w1_ref, b1_ref, g1_ref, be1_ref,
               w2_ref, b2_ref, g2_ref, be2_ref, lw_ref, lb_ref, oin_ref,
               out_ref):
    a = agg_ref[...][:, :H]
    xin = x_ref[...][:, :H] + a[:N] + a[N:]
    h = _mlp(xin, w1_ref[...], b1_ref[...], g1_ref[...], be1_ref[...],
             w2_ref[...], b2_ref[...], g2_ref[...], be2_ref[...])
    pool, _ = _pool(h, b_ref[...])
    out_ref[...] = (oin_ref[...]
                    + jnp.dot(pool, lw_ref[...], preferred_element_type=jnp.float32)
                    + lb_ref[...])


def _pool_first_body(h_ref, b_ref, lw_ref, lb_ref, out_ref):
    pool, cnt = _pool(h_ref[...][:, :H], b_ref[...])
    out_ref[...] = (jnp.dot(pool, lw_ref[...], preferred_element_type=jnp.float32)
                    + cnt * lb_ref[...])


def _pool_body(h_ref, b_ref, lw_ref, lb_ref, oin_ref, out_ref):
    pool, _ = _pool(h_ref[...][:, :H], b_ref[...])
    out_ref[...] = (oin_ref[...]
                    + jnp.dot(pool, lw_ref[...], preferred_element_type=jnp.float32)
                    + lb_ref[...])


def _zeros_body(z_ref):
    z_ref[...] = jnp.zeros((N, HP), jnp.float32)


_make_zeros = pl.pallas_call(
    _zeros_body,
    out_shape=jax.ShapeDtypeStruct((N, HP), jnp.float32),
)


def _full(*s):
    return pl.BlockSpec(s, lambda: (0,) * len(s))


_xhalf = _full(N, HP)
_agghalf = _full(NC * N, HP)
_w = _full(H, H)
_v = _full(1, H)
_lw = _full(H, T)
_lb = _full(1, T)
_bn = _full(1, N)
_ot = _full(G, T)

_mlp_first = pl.pallas_call(
    _first_body,
    in_specs=[_full(N, DF), _full(DF, H), _v, _v, _v, _w, _v, _v, _v],
    out_specs=_full(N, HP),
    out_shape=jax.ShapeDtypeStruct((N, HP), jnp.float32),
)

_mlp_layer = pl.pallas_call(
    _layer_body,
    in_specs=[_xhalf, _agghalf, _w, _v, _v, _v, _w, _v, _v, _v],
    out_specs=_full(N, HP),
    out_shape=jax.ShapeDtypeStruct((N, HP), jnp.float32),
)

_mlp_last = pl.pallas_call(
    _last_body,
    in_specs=[_xhalf, _agghalf, _bn, _w, _v, _v, _v, _w, _v, _v, _v,
              _lw, _lb, _ot],
    out_specs=_ot,
    out_shape=jax.ShapeDtypeStruct((G, T), jnp.float32),
)

_pool_first = pl.pallas_call(
    _pool_first_body,
    in_specs=[_xhalf, _bn, _lw, _lb],
    out_specs=_ot,
    out_shape=jax.ShapeDtypeStruct((G, T), jnp.float32),
)

_pool_next = pl.pallas_call(
    _pool_body,
    in_specs=[_xhalf, _bn, _lw, _lb, _ot],
    out_specs=_ot,
    out_shape=jax.ShapeDtypeStruct((G, T), jnp.float32),
)


def _edge_agg_body(x_hbm, edge_hbm, zero_hbm, out_hbm,
                   sidxs, didxs, tsidx, tdidx, rows, acc,
                   gsems, ssems, dsems, qsems, tsem):
    cid = lax.axis_index("c")
    sid = lax.axis_index("s")
    wid = sid * NC + cid
    ebase = wid * EPW
    BUF_A = list(range(GPC))
    BUF_B = list(range(GPC, NBUF))

    def wait_scatter(k):
        pltpu.make_async_copy(rows[k], acc.at[didxs[k]], ssems[k]).wait()

    def wait_gather(k):
        pltpu.make_async_copy(x_hbm.at[sidxs[k]], rows[k], gsems[k]).wait()

    def wait_idx(ref, sem):
        pltpu.make_async_copy(edge_hbm.at[pl.ds(0, CHUNK)], ref, sem).wait()

    def fire_scatters(bufs):
        for k in bufs:
            wait_idx(didxs[k], dsems[k])
            wait_gather(k)
            pltpu.make_async_copy(rows[k], acc.at[didxs[k]],
                                  ssems[k]).start(add=True)

    def phase(gi, cur, prv):
        @pl.when(gi > 1)
        def _():
            for k in cur:
                wait_scatter(k)

        base = ebase + gi * (GPC * CHUNK)
        for j, k in enumerate(cur):
            off = base + j * CHUNK
            pltpu.async_copy(edge_hbm.at[pl.ds(off, CHUNK)],
                             sidxs[k], qsems[k])
            pltpu.async_copy(edge_hbm.at[pl.ds(E + off, CHUNK)],
                             didxs[k], dsems[k])
        for k in cur:
            wait_idx(sidxs[k], qsems[k])
            pltpu.make_async_copy(x_hbm.at[sidxs[k]], rows[k],
                                  gsems[k]).start()
        @pl.when(gi > 0)
        def _():
            fire_scatters(prv)

    def body(i, carry):
        @pl.when(i % 2 == 0)
        def _():
            phase(i, BUF_A, BUF_B)

        @pl.when(i % 2 == 1)
        def _():
            phase(i, BUF_B, BUF_A)
        return carry

    for j, k in enumerate(BUF_A):
        off = ebase + j * CHUNK
        pltpu.async_copy(edge_hbm.at[pl.ds(off, CHUNK)], sidxs[k], qsems[k])
        pltpu.async_copy(edge_hbm.at[pl.ds(E + off, CHUNK)],
                         didxs[k], dsems[k])
    for k in BUF_A:
        wait_idx(sidxs[k], qsems[k])
        pltpu.make_async_copy(x_hbm.at[sidxs[k]], rows[k], gsems[k]).start()

    pltpu.sync_copy(zero_hbm.at[pl.ds(sid * RPT, RPT)],
                    acc.at[pl.ds(sid * RPT, RPT)])

    @pl.when(sid == NS - 1)
    def _():
        pltpu.sync_copy(zero_hbm.at[pl.ds(NS * RPT, TAIL)],
                        acc.at[pl.ds(NS * RPT, TAIL)])

    plsc.subcore_barrier()

    lax.fori_loop(1, NGRP, body, 0)
    fire_scatters(BUF_B)
    for k in range(NBUF):
        wait_scatter(k)
    toff = ebase + NFULL * CHUNK
    pltpu.sync_copy(edge_hbm.at[pl.ds(toff, TAILE)], tsidx)
    pltpu.sync_copy(edge_hbm.at[pl.ds(E + toff, TAILE)], tdidx)
    pltpu.async_copy(x_hbm.at[tsidx], rows[0].at[pl.ds(0, TAILE)],
                     tsem).wait()
    pltpu.sync_copy(rows[0].at[pl.ds(0, TAILE)], acc.at[tdidx], add=True)
    plsc.subcore_barrier()
    obase = cid * N + sid * RPT
    pltpu.sync_copy(acc.at[pl.ds(sid * RPT, RPT)],
                    out_hbm.at[pl.ds(obase, RPT)])

    @pl.when(sid == NS - 1)
    def _():
        pltpu.sync_copy(acc.at[pl.ds(NS * RPT, TAIL)],
                        out_hbm.at[pl.ds(cid * N + NS * RPT, TAIL)])


@functools.lru_cache(maxsize=None)
def _edge_agg_kernel():
    return pl.kernel(
        _edge_agg_body,
        out_type=jax.ShapeDtypeStruct((NC * N, HP), jnp.float32),
        mesh=plsc.VectorSubcoreMesh(core_axis_name="c", subcore_axis_name="s",
                                    num_cores=NC, num_subcores=NS),
        scratch_types=[
            [pltpu.VMEM((CHUNK,), jnp.int32) for _ in range(NBUF)],
            [pltpu.VMEM((CHUNK,), jnp.int32) for _ in range(NBUF)],
            pltpu.VMEM((TAILE,), jnp.int32),
            pltpu.VMEM((TAILE,), jnp.int32),
            [pltpu.VMEM((CHUNK, HP), jnp.float32) for _ in range(NBUF)],
            pltpu.VMEM_SHARED((N, HP), jnp.float32),
            [pltpu.SemaphoreType.DMA for _ in range(NBUF)],
            [pltpu.SemaphoreType.DMA for _ in range(NBUF)],
            [pltpu.SemaphoreType.DMA for _ in range(NBUF)],
            [pltpu.SemaphoreType.DMA for _ in range(NBUF)],
            pltpu.SemaphoreType.DMA,
        ],
    )


def _edge_agg(x0, eflat, zeros):
    return _edge_agg_kernel()(x0, eflat, zeros)


def kernel(x, edge_index, batch, fh_W1, fh_b1, fh_g1, fh_be1, fh_W2, fh_b2,
           fh_g2, fh_be2, c1_W1, c1_b1, c1_g1, c1_be1, c1_W2, c1_b2, c1_g2,
           c1_be2, c2_W1, c2_b1, c2_g1, c2_be1, c2_W2, c2_b2, c2_g2, c2_be2,
           lin_W0, lin_b0, lin_W1, lin_b1, lin_W2, lin_b2):
    eflat = edge_index.reshape(2 * E)
    brow = batch.reshape(1, N)
    zeros = _make_zeros()
    r = lambda a: a.reshape(1, -1)

    x0 = _mlp_first(x, fh_W1, r(fh_b1), r(fh_g1), r(fh_be1),
                    fh_W2, r(fh_b2), r(fh_g2), r(fh_be2))
    agg = _edge_agg(x0, eflat, zeros)
    out = _pool_first(x0, brow, lin_W0, r(lin_b0))
    x1 = _mlp_layer(x0, agg, c1_W1, r(c1_b1), r(c1_g1), r(c1_be1),
                    c1_W2, r(c1_b2), r(c1_g2), r(c1_be2))
    agg = _edge_agg(x1, eflat, zeros)
    out = _pool_next(x1, brow, lin_W1, r(lin_b1), out)
    out = _mlp_last(x1, agg, brow, c2_W1, r(c2_b1), r(c2_g1), r(c2_be1),
                    c2_W2, r(c2_b2), r(c2_g2), r(c2_be2),
                    lin_W2, r(lin_b2), out)
    return out


# speedup vs baseline: 11.1408x; 1.0115x over previous
import functools

import jax
import jax.numpy as jnp
from jax import lax
from jax.experimental import pallas as pl
from jax.experimental.pallas import tpu as pltpu
from jax.experimental.pallas import tpu_sc as plsc

N = 10000
E = 320000
DF = 128
H = 64
T = 10
G = 64
HP = 128

NC = 2
NS = 16
NW = NC * NS
EPW = E // NW
CHUNK = 64
GPC = 3
NBUF = 2 * GPC
NFULL = EPW // CHUNK
NGRP = NFULL // GPC
TAILE = EPW - NFULL * CHUNK
RPT = 624
TAIL = N - NS * RPT


def _bn_relu(h, g, b):
    m = jnp.mean(h, axis=0, keepdims=True)
    m2 = jnp.mean(h * h, axis=0, keepdims=True)
    v = m2 - m * m
    return jax.nn.relu((h - m) / jnp.sqrt(v + 1e-5) * g + b)


def _mlp(h, w1, b1, g1, be1, w2, b2, g2, be2):
    h = jnp.dot(h, w1, preferred_element_type=jnp.float32) + b1
    h = _bn_relu(h, g1, be1)
    h = jnp.dot(h, w2, preferred_element_type=jnp.float32) + b2
    h = _bn_relu(h, g2, be2)
    return h


def _pool(h, batch_row):
    onehot = (lax.broadcasted_iota(jnp.int32, (G, N), 0) == batch_row)
    onehot = onehot.astype(jnp.float32)
    pool = jnp.dot(onehot, h, preferred_element_type=jnp.float32)
    cnt = jnp.sum(onehot, axis=1, keepdims=True)
    return pool, cnt


def _first_body(x_ref, w1_ref, b1_ref, g1_ref, be1_ref,
                w2_ref, b2_ref, g2_ref, be2_ref, h_ref):
    h = _mlp(x_ref[...], w1_ref[...], b1_ref[...], g1_ref[...], be1_ref[...],
             w2_ref[...], b2_ref[...], g2_ref[...], be2_ref[...])
    h_ref[...] = jnp.concatenate([h, jnp.zeros_like(h)], axis=1)


def _layer_body(agg_ref, w1_ref, b1_ref, g1_ref, be1_ref,
                w2_ref, b2_ref, g2_ref, be2_ref, h_ref):
    a = agg_ref[...][:, :H]
    xin = a[:N] + a[N:]
    h = _mlp(xin, w1_ref[...], b1_ref[...], g1_ref[...], be1_ref[...],
             w2_ref[...], b2_ref[...], g2_ref[...], be2_ref[...])
    h_ref[...] = jnp.concatenate([h, jnp.zeros_like(h)], axis=1)


def _last_body(agg_ref, b_ref, w1_ref, b1_ref, g1_ref, be1_ref,
               w2_ref, b2_ref, g2_ref, be2_ref, lw_ref, lb_ref, oin_ref,
               out_ref):
    a = agg_ref[...][:, :H]
    xin = a[:N] + a[N:]
    h = _mlp(xin, w1_ref[...], b1_ref[...], g1_ref[...], be1_ref[...],
             w2_ref[...], b2_ref[...], g2_ref[...], be2_ref[...])
    pool, _ = _pool(h, b_ref[...])
    out_ref[...] = (oin_ref[...]
                    + jnp.dot(pool, lw_ref[...], preferred_element_type=jnp.float32)
                    + lb_ref[...])


def _pool_first_body(h_ref, b_ref, lw_ref, lb_ref, out_ref):
    pool, cnt = _pool(h_ref[...][:, :H], b_ref[...])
    out_ref[...] = (jnp.dot(pool, lw_ref[...], preferred_element_type=jnp.float32)
                    + cnt * lb_ref[...])


def _pool_body(h_ref, b_ref, lw_ref, lb_ref, oin_ref, out_ref):
    pool, _ = _pool(h_ref[...][:, :H], b_ref[...])
    out_ref[...] = (oin_ref[...]
                    + jnp.dot(pool, lw_ref[...], preferred_element_type=jnp.float32)
                    + lb_ref[...])


def _zeros_body(z_ref):
    z_ref[...] = jnp.zeros((N, HP), jnp.float32)


_make_zeros = pl.pallas_call(
    _zeros_body,
    out_shape=jax.ShapeDtypeStruct((N, HP), jnp.float32),
)


def _full(*s):
    return pl.BlockSpec(s, lambda: (0,) * len(s))


_xhalf = _full(N, HP)
_agghalf = _full(NC * N, HP)
_w = _full(H, H)
_v = _full(1, H)
_lw = _full(H, T)
_lb = _full(1, T)
_bn = _full(1, N)
_ot = _full(G, T)

_mlp_first = pl.pallas_call(
    _first_body,
    in_specs=[_full(N, DF), _full(DF, H), _v, _v, _v, _w, _v, _v, _v],
    out_specs=_full(N, HP),
    out_shape=jax.ShapeDtypeStruct((N, HP), jnp.float32),
)

_mlp_layer = pl.pallas_call(
    _layer_body,
    in_specs=[_agghalf, _w, _v, _v, _v, _w, _v, _v, _v],
    out_specs=_full(N, HP),
    out_shape=jax.ShapeDtypeStruct((N, HP), jnp.float32),
)

_mlp_last = pl.pallas_call(
    _last_body,
    in_specs=[_agghalf, _bn, _w, _v, _v, _v, _w, _v, _v, _v,
              _lw, _lb, _ot],
    out_specs=_ot,
    out_shape=jax.ShapeDtypeStruct((G, T), jnp.float32),
)

_pool_first = pl.pallas_call(
    _pool_first_body,
    in_specs=[_xhalf, _bn, _lw, _lb],
    out_specs=_ot,
    out_shape=jax.ShapeDtypeStruct((G, T), jnp.float32),
)

_pool_next = pl.pallas_call(
    _pool_body,
    in_specs=[_xhalf, _bn, _lw, _lb, _ot],
    out_specs=_ot,
    out_shape=jax.ShapeDtypeStruct((G, T), jnp.float32),
)


def _edge_agg_body(x_hbm, edge_hbm, zero_hbm, out_hbm,
                   sidxs, didxs, tsidx, tdidx, rows, acc,
                   gsems, ssems, dsems, qsems, tsem):
    cid = lax.axis_index("c")
    sid = lax.axis_index("s")
    wid = sid * NC + cid
    ebase = wid * EPW
    BUF_A = list(range(GPC))
    BUF_B = list(range(GPC, NBUF))

    def wait_scatter(k):
        pltpu.make_async_copy(rows[k], acc.at[didxs[k]], ssems[k]).wait()

    def wait_gather(k):
        pltpu.make_async_copy(x_hbm.at[sidxs[k]], rows[k], gsems[k]).wait()

    def wait_idx(ref, sem):
        pltpu.make_async_copy(edge_hbm.at[pl.ds(0, CHUNK)], ref, sem).wait()

    def fire_scatters(bufs):
        for k in bufs:
            wait_idx(didxs[k], dsems[k])
            wait_gather(k)
            pltpu.make_async_copy(rows[k], acc.at[didxs[k]],
                                  ssems[k]).start(add=True)

    def phase(gi, cur, prv):
        @pl.when(gi > 1)
        def _():
            for k in cur:
                wait_scatter(k)

        base = ebase + gi * (GPC * CHUNK)
        for j, k in enumerate(cur):
            off = base + j * CHUNK
            pltpu.async_copy(edge_hbm.at[pl.ds(off, CHUNK)],
                             sidxs[k], qsems[k])
            pltpu.async_copy(edge_hbm.at[pl.ds(E + off, CHUNK)],
                             didxs[k], dsems[k])
        for k in cur:
            wait_idx(sidxs[k], qsems[k])
            pltpu.make_async_copy(x_hbm.at[sidxs[k]], rows[k],
                                  gsems[k]).start()
        @pl.when(gi > 0)
        def _():
            fire_scatters(prv)

    def body(i, carry):
        @pl.when(i % 2 == 0)
        def _():
            phase(i, BUF_A, BUF_B)

        @pl.when(i % 2 == 1)
        def _():
            phase(i, BUF_B, BUF_A)
        return carry

    for j, k in enumerate(BUF_A):
        off = ebase + j * CHUNK
        pltpu.async_copy(edge_hbm.at[pl.ds(off, CHUNK)], sidxs[k], qsems[k])
        pltpu.async_copy(edge_hbm.at[pl.ds(E + off, CHUNK)],
                         didxs[k], dsems[k])
    for k in BUF_A:
        wait_idx(sidxs[k], qsems[k])
        pltpu.make_async_copy(x_hbm.at[sidxs[k]], rows[k], gsems[k]).start()

    @pl.when(cid == 0)
    def _():
        pltpu.sync_copy(x_hbm.at[pl.ds(sid * RPT, RPT)],
                        acc.at[pl.ds(sid * RPT, RPT)])

        @pl.when(sid == NS - 1)
        def _():
            pltpu.sync_copy(x_hbm.at[pl.ds(NS * RPT, TAIL)],
                            acc.at[pl.ds(NS * RPT, TAIL)])

    @pl.when(cid == 1)
    def _():
        pltpu.sync_copy(zero_hbm.at[pl.ds(sid * RPT, RPT)],
                        acc.at[pl.ds(sid * RPT, RPT)])

        @pl.when(sid == NS - 1)
        def _():
            pltpu.sync_copy(zero_hbm.at[pl.ds(NS * RPT, TAIL)],
                            acc.at[pl.ds(NS * RPT, TAIL)])

    plsc.subcore_barrier()

    lax.fori_loop(1, NGRP, body, 0)
    fire_scatters(BUF_B)
    for k in range(NBUF):
        wait_scatter(k)
    toff = ebase + NFULL * CHUNK
    pltpu.sync_copy(edge_hbm.at[pl.ds(toff, TAILE)], tsidx)
    pltpu.sync_copy(edge_hbm.at[pl.ds(E + toff, TAILE)], tdidx)
    pltpu.async_copy(x_hbm.at[tsidx], rows[0].at[pl.ds(0, TAILE)],
                     tsem).wait()
    pltpu.sync_copy(rows[0].at[pl.ds(0, TAILE)], acc.at[tdidx], add=True)
    plsc.subcore_barrier()
    obase = cid * N + sid * RPT
    pltpu.sync_copy(acc.at[pl.ds(sid * RPT, RPT)],
                    out_hbm.at[pl.ds(obase, RPT)])

    @pl.when(sid == NS - 1)
    def _():
        pltpu.sync_copy(acc.at[pl.ds(NS * RPT, TAIL)],
                        out_hbm.at[pl.ds(cid * N + NS * RPT, TAIL)])


@functools.lru_cache(maxsize=None)
def _edge_agg_kernel():
    return pl.kernel(
        _edge_agg_body,
        out_type=jax.ShapeDtypeStruct((NC * N, HP), jnp.float32),
        mesh=plsc.VectorSubcoreMesh(core_axis_name="c", subcore_axis_name="s",
                                    num_cores=NC, num_subcores=NS),
        scratch_types=[
            [pltpu.VMEM((CHUNK,), jnp.int32) for _ in range(NBUF)],
            [pltpu.VMEM((CHUNK,), jnp.int32) for _ in range(NBUF)],
            pltpu.VMEM((TAILE,), jnp.int32),
            pltpu.VMEM((TAILE,), jnp.int32),
            [pltpu.VMEM((CHUNK, HP), jnp.float32) for _ in range(NBUF)],
            pltpu.VMEM_SHARED((N, HP), jnp.float32),
            [pltpu.SemaphoreType.DMA for _ in range(NBUF)],
            [pltpu.SemaphoreType.DMA for _ in range(NBUF)],
            [pltpu.SemaphoreType.DMA for _ in range(NBUF)],
            [pltpu.SemaphoreType.DMA for _ in range(NBUF)],
            pltpu.SemaphoreType.DMA,
        ],
    )


def _edge_agg(x0, eflat, zeros):
    return _edge_agg_kernel()(x0, eflat, zeros)


def kernel(x, edge_index, batch, fh_W1, fh_b1, fh_g1, fh_be1, fh_W2, fh_b2,
           fh_g2, fh_be2, c1_W1, c1_b1, c1_g1, c1_be1, c1_W2, c1_b2, c1_g2,
           c1_be2, c2_W1, c2_b1, c2_g1, c2_be1, c2_W2, c2_b2, c2_g2, c2_be2,
           lin_W0, lin_b0, lin_W1, lin_b1, lin_W2, lin_b2):
    eflat = edge_index.reshape(2 * E)
    brow = batch.reshape(1, N)
    zeros = _make_zeros()
    r = lambda a: a.reshape(1, -1)

    x0 = _mlp_first(x, fh_W1, r(fh_b1), r(fh_g1), r(fh_be1),
                    fh_W2, r(fh_b2), r(fh_g2), r(fh_be2))
    agg = _edge_agg(x0, eflat, zeros)
    out = _pool_first(x0, brow, lin_W0, r(lin_b0))
    x1 = _mlp_layer(agg, c1_W1, r(c1_b1), r(c1_g1), r(c1_be1),
                    c1_W2, r(c1_b2), r(c1_g2), r(c1_be2))
    agg = _edge_agg(x1, eflat, zeros)
    out = _pool_next(x1, brow, lin_W1, r(lin_b1), out)
    out = _mlp_last(agg, brow, c2_W1, r(c2_b1), r(c2_g1), r(c2_be1),
                    c2_W2, r(c2_b2), r(c2_g2), r(c2_be2),
                    lin_W2, r(lin_b2), out)
    return out
